# baseline (speedup 1.0000x reference)
import jax
import jax.numpy as jnp
from jax.experimental import pallas as pl
from jax.experimental.pallas import tpu as pltpu

C = 768
R = 16
E = 4
HID = C // 2

_DN_T = (((1,), (1,)), ((), ()))


def _dot_t(a, b):
    return jax.lax.dot_general(a, b, _DN_T,
                               preferred_element_type=jnp.float32)


def _fused_kernel(x_ref, t_ref, fcw_ref, lns_ref, lnb_ref, wg_ref,
                  fw1_ref, fb1_ref, fw2_ref, fb2_ref,
                  ew1_ref, eb1_ref, ew2_ref, eb2_ref,
                  out_ref, aux_ref, yb_ref, gates_ref, acc_ref, stat_ref):
    b = pl.program_id(0)
    p = pl.program_id(1)
    n = x_ref.shape[1]

    @pl.when((b == 0) & (p == 0))
    def _init():
        stat_ref[...] = jnp.zeros_like(stat_ref)

    @pl.when(p == 0)
    def _phase0():
        x3 = x_ref[0]
        t3 = t_ref[0]

        s = (jnp.sum(x3, axis=0, keepdims=True)
             + jnp.sum(t3, axis=0, keepdims=True)) * (1.0 / n)
        hh = jnp.maximum(_dot_t(s, fw1_ref[...]) + fb1_ref[...], 0.0)
        a = jax.nn.sigmoid(_dot_t(hh, fw2_ref[...]) + fb2_ref[...])

        xb = x3.astype(jnp.bfloat16)
        tb = t3.astype(jnp.bfloat16)
        wx = fcw_ref[:, :C].astype(jnp.bfloat16)
        wt = fcw_ref[:, C:].astype(jnp.bfloat16)
        y = _dot_t(xb, wx) + _dot_t(tb, wt)
        y = jnp.maximum(y, 0.0)

        mu = jnp.mean(y, axis=1, keepdims=True)
        d = y - mu
        var = jnp.mean(d * d, axis=1, keepdims=True)
        y = d / jnp.sqrt(var + 1e-5) * lns_ref[...] + lnb_ref[...]
        yb = y.astype(jnp.bfloat16)
        yb_ref[...] = yb

        lg = jnp.dot(yb, wg_ref[...].astype(jnp.bfloat16),
                     preferred_element_type=jnp.float32)
        ii = jax.lax.broadcasted_iota(jnp.int32, lg.shape, 1)
        m1 = jnp.max(lg, axis=1, keepdims=True)
        i1 = jnp.min(jnp.where(lg == m1, ii, E), axis=1, keepdims=True)
        one1 = ii == i1
        lg2 = jnp.where(one1, -jnp.inf, lg)
        m2 = jnp.max(lg2, axis=1, keepdims=True)
        i2 = jnp.min(jnp.where(lg2 == m2, ii, E), axis=1, keepdims=True)
        one2 = ii == i2
        e21 = jnp.exp(m2 - m1)
        g1 = 1.0 / (1.0 + e21)
        g2 = e21 * g1
        gates = jnp.where(one1, g1, 0.0) + jnp.where(one2, g2, 0.0)
        gates_ref[...] = gates

        stat_ref[0:1, 0:E] += jnp.sum(gates, axis=0, keepdims=True)
        stat_ref[1:2, 0:E] += jnp.sum((gates > 0.0).astype(jnp.float32),
                                      axis=0, keepdims=True)

        acc_ref[...] = x3 * a + t3 * (1.0 - a)

    @pl.when(p > 0)
    def _expert():
        w1 = ew1_ref[0].astype(jnp.bfloat16)
        h = _dot_t(yb_ref[...], w1) + eb1_ref[0]
        h = jnp.maximum(h, 0.0)
        hb = h.astype(jnp.bfloat16)
        w2 = ew2_ref[0].astype(jnp.bfloat16)
        eo = _dot_t(hb, w2) + eb2_ref[0]
        ge = jax.lax.switch(
            p - 1, [lambda i=i: gates_ref[:, i:i + 1] for i in range(E)])
        acc_ref[...] += ge * eo

    @pl.when(p == E)
    def _emit():
        out_ref[0] = acc_ref[...]

    @pl.when((b == pl.num_programs(0) - 1) & (p == E))
    def _fin():
        imp = stat_ref[0:1, 0:E]
        mi = jnp.mean(imp)
        vi = jnp.mean((imp - mi) ** 2)
        ld = stat_ref[1:2, 0:E]
        ml = jnp.mean(ld)
        vl = jnp.mean((ld - ml) ** 2)
        aux = (vi / (mi * mi + 1e-10) + vl / (ml * ml + 1e-10)) * 1e-2
        aux_ref[...] = jnp.reshape(aux, (1, 1))


def _expert_ix(b, p):
    e = jnp.maximum(p - 1, 0)
    return (e, 0, 0)


def _expert_ix2(b, p):
    e = jnp.maximum(p - 1, 0)
    return (e, 0)


def kernel(x, t, fc_w, ln_scale, ln_bias, f_w1, f_b1, f_w2, f_b2,
           w_gate, e_w1, e_b1, e_w2, e_b2, task_index):
    B, Cx, H, W = x.shape
    N = H * W
    x3 = jnp.transpose(x, (0, 2, 3, 1)).reshape(B, N, Cx)
    t3 = jnp.transpose(t, (0, 2, 3, 1)).reshape(B, N, Cx)
    wg = jax.lax.dynamic_index_in_dim(w_gate, task_index, 0,
                                      keepdims=False)

    out3, aux = pl.pallas_call(
        _fused_kernel,
        grid=(B, E + 1),
        in_specs=[
            pl.BlockSpec((1, N, Cx), lambda b, p: (b, 0, 0)),
            pl.BlockSpec((1, N, Cx), lambda b, p: (b, 0, 0)),
            pl.BlockSpec((Cx, 2 * Cx), lambda b, p: (0, 0)),
            pl.BlockSpec((1, Cx), lambda b, p: (0, 0)),
            pl.BlockSpec((1, Cx), lambda b, p: (0, 0)),
            pl.BlockSpec((Cx, E), lambda b, p: (0, 0)),
            pl.BlockSpec((Cx // R, Cx), lambda b, p: (0, 0)),
            pl.BlockSpec((1, Cx // R), lambda b, p: (0, 0)),
            pl.BlockSpec((Cx, Cx // R), lambda b, p: (0, 0)),
            pl.BlockSpec((1, Cx), lambda b, p: (0, 0)),
            pl.BlockSpec((1, HID, Cx), _expert_ix),
            pl.BlockSpec((1, 1, HID), _expert_ix),
            pl.BlockSpec((1, Cx, HID), _expert_ix),
            pl.BlockSpec((1, 1, Cx), _expert_ix),
        ],
        out_specs=[
            pl.BlockSpec((1, N, Cx), lambda b, p: (b, 0, 0)),
            pl.BlockSpec((1, 1), lambda b, p: (0, 0)),
        ],
        out_shape=[
            jax.ShapeDtypeStruct((B, N, Cx), jnp.float32),
            jax.ShapeDtypeStruct((1, 1), jnp.float32),
        ],
        scratch_shapes=[
            pltpu.VMEM((N, Cx), jnp.bfloat16),
            pltpu.VMEM((N, E), jnp.float32),
            pltpu.VMEM((N, Cx), jnp.float32),
            pltpu.VMEM((8, 128), jnp.float32),
        ],
    )(x3, t3, fc_w, ln_scale.reshape(1, Cx), ln_bias.reshape(1, Cx), wg,
      f_w1, f_b1.reshape(1, Cx // R), f_w2, f_b2.reshape(1, Cx),
      e_w1, e_b1.reshape(E, 1, HID), e_w2, e_b2.reshape(E, 1, Cx))

    out = jnp.transpose(out3.reshape(B, H, W, Cx), (0, 3, 1, 2))
    return out, aux.reshape(())


# speedup vs baseline: 1.2201x; 1.2201x over previous
import jax
import jax.numpy as jnp
from jax.experimental import pallas as pl
from jax.experimental.pallas import tpu as pltpu

C = 768
R = 16
E = 4
HID = C // 2

_DN_T = (((1,), (1,)), ((), ()))


def _dot_t(a, b):
    return jax.lax.dot_general(a, b, _DN_T,
                               preferred_element_type=jnp.float32)


def _fused_kernel(x_ref, t_ref, fcw_ref, lns_ref, lnb_ref, wg_ref,
                  fw1_ref, fb1_ref, fw2_ref, fb2_ref,
                  ew1_ref, eb1_ref, ew2_ref, eb2_ref,
                  out_ref, aux_ref, acc_ref):
    b = pl.program_id(0)
    n = x_ref.shape[1]

    @pl.when(b == 0)
    def _init():
        acc_ref[...] = jnp.zeros_like(acc_ref)

    x3 = x_ref[0]
    t3 = t_ref[0]

    s = (jnp.sum(x3, axis=0, keepdims=True)
         + jnp.sum(t3, axis=0, keepdims=True)) * (1.0 / n)
    hh = jnp.maximum(_dot_t(s, fw1_ref[...]) + fb1_ref[...], 0.0)
    a = jax.nn.sigmoid(_dot_t(hh, fw2_ref[...]) + fb2_ref[...])

    xb = x3.astype(jnp.bfloat16)
    tb = t3.astype(jnp.bfloat16)
    wx = fcw_ref[:, :C].astype(jnp.bfloat16)
    wt = fcw_ref[:, C:].astype(jnp.bfloat16)
    y = _dot_t(xb, wx) + _dot_t(tb, wt)
    y = jnp.maximum(y, 0.0)

    mu = jnp.mean(y, axis=1, keepdims=True)
    d = y - mu
    var = jnp.mean(d * d, axis=1, keepdims=True)
    y = d / jnp.sqrt(var + 1e-5) * lns_ref[...] + lnb_ref[...]
    yb = y.astype(jnp.bfloat16)

    lg = jnp.dot(yb, wg_ref[...].astype(jnp.bfloat16),
                 preferred_element_type=jnp.float32)
    ii = jax.lax.broadcasted_iota(jnp.int32, lg.shape, 1)
    m1 = jnp.max(lg, axis=1, keepdims=True)
    i1 = jnp.min(jnp.where(lg == m1, ii, E), axis=1, keepdims=True)
    one1 = ii == i1
    lg2 = jnp.where(one1, -jnp.inf, lg)
    m2 = jnp.max(lg2, axis=1, keepdims=True)
    i2 = jnp.min(jnp.where(lg2 == m2, ii, E), axis=1, keepdims=True)
    one2 = ii == i2
    e21 = jnp.exp(m2 - m1)
    g1 = 1.0 / (1.0 + e21)
    g2 = e21 * g1
    gates = jnp.where(one1, g1, 0.0) + jnp.where(one2, g2, 0.0)

    imp = jnp.sum(gates, axis=0, keepdims=True)
    ld = jnp.sum((gates > 0.0).astype(jnp.float32), axis=0, keepdims=True)
    acc_ref[0:1, 0:E] += imp
    acc_ref[1:2, 0:E] += ld

    acc = x3 * a + t3 * (1.0 - a)
    for e in range(E):
        w1 = ew1_ref[e].astype(jnp.bfloat16)
        h = _dot_t(yb, w1) + eb1_ref[e][None, :]
        h = jnp.maximum(h, 0.0)
        hb = h.astype(jnp.bfloat16)
        w2 = ew2_ref[e].astype(jnp.bfloat16)
        eo = _dot_t(hb, w2) + eb2_ref[e][None, :]
        acc = acc + gates[:, e:e + 1] * eo

    out_ref[0] = acc

    @pl.when(b == pl.num_programs(0) - 1)
    def _fin():
        imp = acc_ref[0:1, 0:E]
        mi = jnp.mean(imp)
        vi = jnp.mean((imp - mi) ** 2)
        ld = acc_ref[1:2, 0:E]
        ml = jnp.mean(ld)
        vl = jnp.mean((ld - ml) ** 2)
        aux = (vi / (mi * mi + 1e-10) + vl / (ml * ml + 1e-10)) * 1e-2
        aux_ref[...] = jnp.reshape(aux, (1, 1))


def kernel(x, t, fc_w, ln_scale, ln_bias, f_w1, f_b1, f_w2, f_b2,
           w_gate, e_w1, e_b1, e_w2, e_b2, task_index):
    B, Cx, H, W = x.shape
    N = H * W
    x3 = jnp.transpose(x, (0, 2, 3, 1)).reshape(B, N, Cx)
    t3 = jnp.transpose(t, (0, 2, 3, 1)).reshape(B, N, Cx)
    wg = jax.lax.dynamic_index_in_dim(w_gate, task_index, 0,
                                      keepdims=False)

    out3, aux = pl.pallas_call(
        _fused_kernel,
        grid=(B,),
        in_specs=[
            pl.BlockSpec((1, N, Cx), lambda b: (b, 0, 0)),
            pl.BlockSpec((1, N, Cx), lambda b: (b, 0, 0)),
            pl.BlockSpec((Cx, 2 * Cx), lambda b: (0, 0)),
            pl.BlockSpec((1, Cx), lambda b: (0, 0)),
            pl.BlockSpec((1, Cx), lambda b: (0, 0)),
            pl.BlockSpec((Cx, E), lambda b: (0, 0)),
            pl.BlockSpec((Cx // R, Cx), lambda b: (0, 0)),
            pl.BlockSpec((1, Cx // R), lambda b: (0, 0)),
            pl.BlockSpec((Cx, Cx // R), lambda b: (0, 0)),
            pl.BlockSpec((1, Cx), lambda b: (0, 0)),
            pl.BlockSpec((E, HID, Cx), lambda b: (0, 0, 0)),
            pl.BlockSpec((E, HID), lambda b: (0, 0)),
            pl.BlockSpec((E, Cx, HID), lambda b: (0, 0, 0)),
            pl.BlockSpec((E, Cx), lambda b: (0, 0)),
        ],
        out_specs=[
            pl.BlockSpec((1, N, Cx), lambda b: (b, 0, 0)),
            pl.BlockSpec((1, 1), lambda b: (0, 0)),
        ],
        out_shape=[
            jax.ShapeDtypeStruct((B, N, Cx), jnp.float32),
            jax.ShapeDtypeStruct((1, 1), jnp.float32),
        ],
        scratch_shapes=[
            pltpu.VMEM((8, 128), jnp.float32),
        ],
    )(x3, t3, fc_w, ln_scale.reshape(1, Cx), ln_bias.reshape(1, Cx), wg,
      f_w1, f_b1.reshape(1, Cx // R), f_w2, f_b2.reshape(1, Cx),
      e_w1, e_b1, e_w2, e_b2)

    out = jnp.transpose(out3.reshape(B, H, W, Cx), (0, 3, 1, 2))
    return out, aux.reshape(())


# speedup vs baseline: 1.2479x; 1.0228x over previous
import jax
import jax.numpy as jnp
from jax.experimental import pallas as pl
from jax.experimental.pallas import tpu as pltpu

C = 768
R = 16
E = 4
HID = C // 2

_DN_T = (((1,), (1,)), ((), ()))


def _dot_t(a, b):
    return jax.lax.dot_general(a, b, _DN_T,
                               preferred_element_type=jnp.float32)


def _fused_kernel(x_ref, t_ref, fcw_ref, lns_ref, lnb_ref, wg_ref,
                  fw1_ref, fb1_ref, fw2_ref, fb2_ref,
                  ew1_hbm, eb1_ref, ew2_hbm, eb2_ref,
                  out_ref, aux_ref, ew1_ref, ew2_ref, acc_ref, sem1, sem2):
    b = pl.program_id(0)
    n = x_ref.shape[1]

    @pl.when(b == 0)
    def _init():
        acc_ref[...] = jnp.zeros_like(acc_ref)
        pltpu.make_async_copy(ew1_hbm, ew1_ref, sem1).start()
        pltpu.make_async_copy(ew2_hbm, ew2_ref, sem2).start()

    x3 = x_ref[0]
    t3 = t_ref[0]

    s = (jnp.sum(x3, axis=0, keepdims=True)
         + jnp.sum(t3, axis=0, keepdims=True)) * (1.0 / n)
    hh = jnp.maximum(_dot_t(s, fw1_ref[...]) + fb1_ref[...], 0.0)
    a = jax.nn.sigmoid(_dot_t(hh, fw2_ref[...]) + fb2_ref[...])

    xb = x3.astype(jnp.bfloat16)
    tb = t3.astype(jnp.bfloat16)
    wx = fcw_ref[:, :C].astype(jnp.bfloat16)
    wt = fcw_ref[:, C:].astype(jnp.bfloat16)
    y = _dot_t(xb, wx) + _dot_t(tb, wt)
    y = jnp.maximum(y, 0.0)

    mu = jnp.mean(y, axis=1, keepdims=True)
    d = y - mu
    var = jnp.mean(d * d, axis=1, keepdims=True)
    y = d / jnp.sqrt(var + 1e-5) * lns_ref[...] + lnb_ref[...]
    yb = y.astype(jnp.bfloat16)

    lg = jnp.dot(yb, wg_ref[...].astype(jnp.bfloat16),
                 preferred_element_type=jnp.float32)
    ii = jax.lax.broadcasted_iota(jnp.int32, lg.shape, 1)
    m1 = jnp.max(lg, axis=1, keepdims=True)
    i1 = jnp.min(jnp.where(lg == m1, ii, E), axis=1, keepdims=True)
    one1 = ii == i1
    lg2 = jnp.where(one1, -jnp.inf, lg)
    m2 = jnp.max(lg2, axis=1, keepdims=True)
    i2 = jnp.min(jnp.where(lg2 == m2, ii, E), axis=1, keepdims=True)
    one2 = ii == i2
    e21 = jnp.exp(m2 - m1)
    g1 = 1.0 / (1.0 + e21)
    g2 = e21 * g1
    gates = jnp.where(one1, g1, 0.0) + jnp.where(one2, g2, 0.0)

    imp = jnp.sum(gates, axis=0, keepdims=True)
    ld = jnp.sum((gates > 0.0).astype(jnp.float32), axis=0, keepdims=True)
    acc_ref[0:1, 0:E] += imp
    acc_ref[1:2, 0:E] += ld

    @pl.when(b == 0)
    def _wait():
        pltpu.make_async_copy(ew1_hbm, ew1_ref, sem1).wait()
        pltpu.make_async_copy(ew2_hbm, ew2_ref, sem2).wait()

    acc = x3 * a + t3 * (1.0 - a)
    for e in range(E):
        w1 = ew1_ref[e].astype(jnp.bfloat16)
        h = _dot_t(yb, w1) + eb1_ref[e][None, :]
        h = jnp.maximum(h, 0.0)
        hb = h.astype(jnp.bfloat16)
        w2 = ew2_ref[e].astype(jnp.bfloat16)
        eo = _dot_t(hb, w2) + eb2_ref[e][None, :]
        acc = acc + gates[:, e:e + 1] * eo

    out_ref[0] = acc

    @pl.when(b == pl.num_programs(0) - 1)
    def _fin():
        imp = acc_ref[0:1, 0:E]
        mi = jnp.mean(imp)
        vi = jnp.mean((imp - mi) ** 2)
        ld = acc_ref[1:2, 0:E]
        ml = jnp.mean(ld)
        vl = jnp.mean((ld - ml) ** 2)
        aux = (vi / (mi * mi + 1e-10) + vl / (ml * ml + 1e-10)) * 1e-2
        aux_ref[...] = jnp.reshape(aux, (1, 1))


def kernel(x, t, fc_w, ln_scale, ln_bias, f_w1, f_b1, f_w2, f_b2,
           w_gate, e_w1, e_b1, e_w2, e_b2, task_index):
    B, Cx, H, W = x.shape
    N = H * W
    x3 = jnp.transpose(x, (0, 2, 3, 1)).reshape(B, N, Cx)
    t3 = jnp.transpose(t, (0, 2, 3, 1)).reshape(B, N, Cx)
    wg = jax.lax.dynamic_index_in_dim(w_gate, task_index, 0,
                                      keepdims=False)

    out3, aux = pl.pallas_call(
        _fused_kernel,
        grid=(B,),
        in_specs=[
            pl.BlockSpec((1, N, Cx), lambda b: (b, 0, 0)),
            pl.BlockSpec((1, N, Cx), lambda b: (b, 0, 0)),
            pl.BlockSpec((Cx, 2 * Cx), lambda b: (0, 0)),
            pl.BlockSpec((1, Cx), lambda b: (0, 0)),
            pl.BlockSpec((1, Cx), lambda b: (0, 0)),
            pl.BlockSpec((Cx, E), lambda b: (0, 0)),
            pl.BlockSpec((Cx // R, Cx), lambda b: (0, 0)),
            pl.BlockSpec((1, Cx // R), lambda b: (0, 0)),
            pl.BlockSpec((Cx, Cx // R), lambda b: (0, 0)),
            pl.BlockSpec((1, Cx), lambda b: (0, 0)),
            pl.BlockSpec(memory_space=pl.ANY),
            pl.BlockSpec((E, HID), lambda b: (0, 0)),
            pl.BlockSpec(memory_space=pl.ANY),
            pl.BlockSpec((E, Cx), lambda b: (0, 0)),
        ],
        out_specs=[
            pl.BlockSpec((1, N, Cx), lambda b: (b, 0, 0)),
            pl.BlockSpec((1, 1), lambda b: (0, 0)),
        ],
        out_shape=[
            jax.ShapeDtypeStruct((B, N, Cx), jnp.float32),
            jax.ShapeDtypeStruct((1, 1), jnp.float32),
        ],
        scratch_shapes=[
            pltpu.VMEM((E, HID, Cx), jnp.float32),
            pltpu.VMEM((E, Cx, HID), jnp.float32),
            pltpu.VMEM((8, 128), jnp.float32),
            pltpu.SemaphoreType.DMA,
            pltpu.SemaphoreType.DMA,
        ],
    )(x3, t3, fc_w, ln_scale.reshape(1, Cx), ln_bias.reshape(1, Cx), wg,
      f_w1, f_b1.reshape(1, Cx // R), f_w2, f_b2.reshape(1, Cx),
      e_w1, e_b1, e_w2, e_b2)

    out = jnp.transpose(out3.reshape(B, H, W, Cx), (0, 3, 1, 2))
    return out, aux.reshape(())
